# Initial kernel scaffold; baseline (speedup 1.0000x reference)
import functools

import jax
import jax.numpy as jnp
from jax import lax
from jax.experimental import pallas as pl
from jax.experimental.pallas import tpu as pltpu
from jax.experimental.pallas import tpu_sc as plsc

N = 10000
E = 320000
D = 128
DF = 8
NSUB = 16
NCORE = 2
SB = 128
CHUNK = 10
ROWS_PER_CORE = E // NCORE // SB
NCHUNK = ROWS_PER_CORE // CHUNK
ZITER = N // 16

R = 200
GRID = N // R


def _sc_aggregate(xflat, src2d, dst2d):
    mesh = plsc.VectorSubcoreMesh(core_axis_name="c", subcore_axis_name="s")

    @functools.partial(
        pl.kernel,
        out_type=[
            jax.ShapeDtypeStruct((NCORE, N, D), jnp.float32),
            jax.ShapeDtypeStruct((NCORE, N), jnp.float32),
        ],
        mesh=mesh,
        scratch_types=[
            pltpu.VMEM((N, DF), jnp.float32),
            pltpu.VMEM((N,), jnp.float32),
            pltpu.VMEM((CHUNK, SB), jnp.int32),
            pltpu.VMEM((CHUNK, SB), jnp.int32),
            pltpu.VMEM((CHUNK, SB), jnp.int32),
            pltpu.VMEM((CHUNK, SB, DF), jnp.float32),
            pltpu.SemaphoreType.DMA,
        ],
    )
    def body(x_hbm, src_hbm, dst_hbm, agg_hbm, cnt_hbm,
             acc_v, cnt_v, src_v, dst_v, idx_v, rows_v, sem):
        c = lax.axis_index("c")
        s = lax.axis_index("s")
        iota = lax.iota(jnp.int32, 16)
        zeros = jnp.zeros((16,), jnp.float32)
        ones = jnp.ones((16,), jnp.float32)

        def zbody(r, _):
            rows = r * 16 + iota
            for j in range(DF):
                plsc.store_scatter(acc_v, [rows, jnp.full((16,), j, jnp.int32)], zeros)
            cnt_v[pl.ds(r * 16, 16)] = zeros
            return _

        lax.fori_loop(0, ZITER, zbody, None)

        def chunk_body(ch, _):
            row0 = c * ROWS_PER_CORE + ch * CHUNK
            pltpu.sync_copy(src_hbm.at[pl.ds(row0, CHUNK)], src_v)
            pltpu.sync_copy(dst_hbm.at[pl.ds(row0, CHUNK)], dst_v)
            for i in range(CHUNK):
                for m in range(SB // 16):
                    v = src_v[i, pl.ds(m * 16, 16)]
                    idx_v[i, pl.ds(m * 16, 16)] = v * NSUB + s
            descs = [
                pltpu.async_copy(x_hbm.at[idx_v.at[i]], rows_v.at[i], sem)
                for i in range(CHUNK)
            ]
            for dsc in descs:
                dsc.wait()

            def compute_i(i, _):
                ii = jnp.full((16,), i, jnp.int32)
                for k in range(SB // 16):
                    dst16 = dst_v[i, pl.ds(k * 16, 16)]
                    rowpat = k * 16 + iota
                    for j in range(DF):
                        jj = jnp.full((16,), j, jnp.int32)
                        data = plsc.load_gather(rows_v, [ii, rowpat, jj])
                        plsc.addupdate_scatter(acc_v, [dst16, jj], data)
                    plsc.addupdate_scatter(cnt_v, [dst16], ones)
                return _

            lax.fori_loop(0, CHUNK, compute_i, None)
            return _

        lax.fori_loop(0, NCHUNK, chunk_body, None)

        pltpu.sync_copy(acc_v, agg_hbm.at[c, :, pl.ds(s * DF, DF)])

        @pl.when(s == 0)
        def _():
            pltpu.sync_copy(cnt_v, cnt_hbm.at[c])

    return body(xflat, src2d, dst2d)


def _tc_combine(agg, cnt, x, W_l, b_l, W_r):
    def body(agg_ref, cnt_ref, x_ref, wl_ref, wr_ref, b_ref, out_ref):
        i = pl.program_id(0)
        a = agg_ref[0] + agg_ref[1]
        cb = cnt_ref[0, pl.ds(i * R, R)] + cnt_ref[1, pl.ds(i * R, R)]
        inv = 1.0 / jnp.maximum(cb, 1.0)
        mean = a * inv[:, None]
        dn = (((1,), (1,)), ((), ()))
        acc = lax.dot_general(mean, wl_ref[...], dn,
                              preferred_element_type=jnp.float32)
        acc = acc + lax.dot_general(x_ref[...], wr_ref[...], dn,
                                    preferred_element_type=jnp.float32)
        out_ref[...] = acc + b_ref[...]

    return pl.pallas_call(
        body,
        grid=(GRID,),
        in_specs=[
            pl.BlockSpec((NCORE, R, D), lambda i: (0, i, 0)),
            pl.BlockSpec((NCORE, N), lambda i: (0, 0)),
            pl.BlockSpec((R, D), lambda i: (i, 0)),
            pl.BlockSpec((D, D), lambda i: (0, 0)),
            pl.BlockSpec((D, D), lambda i: (0, 0)),
            pl.BlockSpec((1, D), lambda i: (0, 0)),
        ],
        out_specs=pl.BlockSpec((R, D), lambda i: (i, 0)),
        out_shape=jax.ShapeDtypeStruct((N, D), jnp.float32),
    )(agg, cnt, x, W_l, W_r, b_l)


def kernel(x, edge_index, W_l, b_l, W_r):
    ei = edge_index.astype(jnp.int32)
    src2d = ei[0].reshape(E // SB, SB)
    dst2d = ei[1].reshape(E // SB, SB)
    xflat = x.reshape(N * NSUB, DF)
    agg, cnt = _sc_aggregate(xflat, src2d, dst2d)
    return _tc_combine(agg, cnt, x, W_l, b_l.reshape(1, D), W_r)


# speedup vs baseline: 1.9192x; 1.9192x over previous
import functools

import jax
import jax.numpy as jnp
from jax import lax
from jax.experimental import pallas as pl
from jax.experimental.pallas import tpu as pltpu
from jax.experimental.pallas import tpu_sc as plsc

N = 10000
E = 320000
D = 128
DF = 8
NSUB = 16
NCORE = 2
SB = 128
CHUNK = 10
ROWS_PER_CORE = E // NCORE // SB
NCHUNK = ROWS_PER_CORE // CHUNK
ZITER = N // 16

R = 200
GRID = N // R


def _sc_aggregate(xflat, src2d, dst2d):
    mesh = plsc.VectorSubcoreMesh(core_axis_name="c", subcore_axis_name="s")

    @functools.partial(
        pl.kernel,
        out_type=[
            jax.ShapeDtypeStruct((NCORE, N, D), jnp.float32),
            jax.ShapeDtypeStruct((NCORE, N), jnp.float32),
        ],
        mesh=mesh,
        compiler_params=pltpu.CompilerParams(use_tc_tiling_on_sc=False,
                                             needs_layout_passes=False),
        scratch_types=[
            pltpu.VMEM((N, DF), jnp.float32),
            pltpu.VMEM((N,), jnp.float32),
            pltpu.VMEM((CHUNK, SB), jnp.int32),
            pltpu.VMEM((CHUNK, SB), jnp.int32),
            pltpu.VMEM((CHUNK, SB), jnp.int32),
            pltpu.VMEM((CHUNK, SB, DF), jnp.float32),
            pltpu.SemaphoreType.DMA,
        ],
    )
    def body(x_hbm, src_hbm, dst_hbm, agg_hbm, cnt_hbm,
             acc_v, cnt_v, src_v, dst_v, idx_v, rows_v, sem):
        c = lax.axis_index("c")
        s = lax.axis_index("s")
        iota = lax.iota(jnp.int32, 16)
        zeros = jnp.zeros((16,), jnp.float32)
        ones = jnp.ones((16,), jnp.float32)

        def zbody(r, _):
            rows = r * 16 + iota
            for j in range(DF):
                plsc.store_scatter(acc_v, [rows, jnp.full((16,), j, jnp.int32)], zeros)
            cnt_v[pl.ds(r * 16, 16)] = zeros
            return _

        lax.fori_loop(0, ZITER, zbody, None)

        def chunk_body(ch, _):
            row0 = c * ROWS_PER_CORE + ch * CHUNK
            pltpu.sync_copy(src_hbm.at[pl.ds(row0, CHUNK)], src_v)
            pltpu.sync_copy(dst_hbm.at[pl.ds(row0, CHUNK)], dst_v)
            for i in range(CHUNK):
                for m in range(SB // 16):
                    v = src_v[i, pl.ds(m * 16, 16)]
                    idx_v[i, pl.ds(m * 16, 16)] = v * NSUB + s
            descs = [
                pltpu.async_copy(x_hbm.at[idx_v.at[i]], rows_v.at[i], sem)
                for i in range(CHUNK)
            ]
            for dsc in descs:
                dsc.wait()

            def compute_i(i, _):
                ii = jnp.full((16,), i, jnp.int32)
                for k in range(SB // 16):
                    dst16 = dst_v[i, pl.ds(k * 16, 16)]
                    rowpat = k * 16 + iota
                    for j in range(DF):
                        jj = jnp.full((16,), j, jnp.int32)
                        data = plsc.load_gather(rows_v, [ii, rowpat, jj])
                        plsc.addupdate_scatter(acc_v, [dst16, jj], data)
                    plsc.addupdate_scatter(cnt_v, [dst16], ones)
                return _

            lax.fori_loop(0, CHUNK, compute_i, None)
            return _

        lax.fori_loop(0, NCHUNK, chunk_body, None)

        pltpu.sync_copy(acc_v, agg_hbm.at[c, :, pl.ds(s * DF, DF)])

        @pl.when(s == 0)
        def _():
            pltpu.sync_copy(cnt_v, cnt_hbm.at[c])

    return body(xflat, src2d, dst2d)


def _tc_combine(agg, cnt, x, W_l, b_l, W_r):
    def body(agg_ref, cnt_ref, x_ref, wl_ref, wr_ref, b_ref, out_ref):
        a = agg_ref[0] + agg_ref[1]
        cb = cnt_ref[0, 0] + cnt_ref[0, 1]
        inv = 1.0 / jnp.maximum(cb, 1.0)
        mean = a * inv[:, None]
        dn = (((1,), (1,)), ((), ()))
        acc = lax.dot_general(mean, wl_ref[...], dn,
                              preferred_element_type=jnp.float32)
        acc = acc + lax.dot_general(x_ref[...], wr_ref[...], dn,
                                    preferred_element_type=jnp.float32)
        out_ref[...] = acc + b_ref[...]

    return pl.pallas_call(
        body,
        grid=(GRID,),
        in_specs=[
            pl.BlockSpec((NCORE, R, D), lambda i: (0, i, 0)),
            pl.BlockSpec((1, NCORE, R), lambda i: (i, 0, 0)),
            pl.BlockSpec((R, D), lambda i: (i, 0)),
            pl.BlockSpec((D, D), lambda i: (0, 0)),
            pl.BlockSpec((D, D), lambda i: (0, 0)),
            pl.BlockSpec((1, D), lambda i: (0, 0)),
        ],
        out_specs=pl.BlockSpec((R, D), lambda i: (i, 0)),
        out_shape=jax.ShapeDtypeStruct((N, D), jnp.float32),
    )(agg, cnt.reshape(NCORE, GRID, R).transpose(1, 0, 2), x, W_l, W_r, b_l)


def kernel(x, edge_index, W_l, b_l, W_r):
    ei = edge_index.astype(jnp.int32)
    src2d = ei[0].reshape(E // SB, SB)
    dst2d = ei[1].reshape(E // SB, SB)
    xflat = x.reshape(N * NSUB, DF)
    agg, cnt = _sc_aggregate(xflat, src2d, dst2d)
    return _tc_combine(agg, cnt, x, W_l, b_l.reshape(1, D), W_r)


# speedup vs baseline: 2.9041x; 1.5132x over previous
import functools

import jax
import jax.numpy as jnp
from jax import lax
from jax.experimental import pallas as pl
from jax.experimental.pallas import tpu as pltpu
from jax.experimental.pallas import tpu_sc as plsc

N = 10000
E = 320000
D = 128
HALF_N = N // 2
DF = 16
NSUB = 16
NCORE = 2
SB = 128
SEG = 25
CH = 5
NQ = SEG // CH
ROWS_PER_CORE = E // NCORE // SB
NS = ROWS_PER_CORE // SEG
IBYTES = 2 * SEG * SB * 4
GBYTES = CH * SB * DF * 4

R = 200
GRID = N // R


def _sc_aggregate(xg, src2d, dst2d):
    mesh = plsc.VectorSubcoreMesh(core_axis_name="c", subcore_axis_name="s")

    @functools.partial(
        pl.kernel,
        out_type=[
            jax.ShapeDtypeStruct((NCORE, N, D), jnp.float32),
            jax.ShapeDtypeStruct((NCORE, 2, HALF_N), jnp.float32),
        ],
        mesh=mesh,
        compiler_params=pltpu.CompilerParams(use_tc_tiling_on_sc=False,
                                             needs_layout_passes=False),
        scratch_types=[
            pltpu.VMEM((HALF_N, DF), jnp.float32),
            pltpu.VMEM((5120,), jnp.float32),
            pltpu.VMEM((2, SEG, SB), jnp.int32),
            pltpu.VMEM((2, SEG, SB), jnp.int32),
            pltpu.VMEM((2, CH, SB, DF), jnp.float32),
            pltpu.SemaphoreType.DMA,
            pltpu.SemaphoreType.DMA,
            pltpu.SemaphoreType.DMA,
            pltpu.SemaphoreType.DMA,
        ],
    )
    def body(x_hbm, src_hbm, dst_hbm, agg_hbm, cnt_hbm,
             acc_v, cnt_v, src_v, dst_v, rows_v, isem0, isem1, rsem0, rsem1):
        c = lax.axis_index("c")
        s = lax.axis_index("s")
        nh = s // 8
        dc = s % 8
        lo16 = jnp.full((16,), nh * HALF_N, jnp.int32)
        iota = lax.iota(jnp.int32, 16)
        zeros = jnp.zeros((16,), jnp.float32)
        ones = jnp.ones((16,), jnp.float32)
        isems = (isem0, isem1)
        rsems = (rsem0, rsem1)

        def zacc(r, carry):
            acc_v[r, :] = zeros
            return carry

        lax.fori_loop(0, HALF_N, zacc, None)

        def zcnt(r, carry):
            cnt_v[pl.ds(r * 16, 16)] = zeros
            return carry

        lax.fori_loop(0, 5120 // 16, zcnt, None)

        def fire_idx(seg, buf):
            r0 = c * ROWS_PER_CORE + seg * SEG
            pltpu.async_copy(src_hbm.at[pl.ds(r0, SEG)], src_v.at[buf],
                             isems[buf])
            pltpu.async_copy(dst_hbm.at[pl.ds(r0, SEG)], dst_v.at[buf],
                             isems[buf])

        def scale(buf):
            pltpu.make_async_copy(src_hbm.at[pl.ds(0, SEG)], src_v.at[buf],
                                  isems[buf]).wait()
            pltpu.make_async_copy(dst_hbm.at[pl.ds(0, SEG)], dst_v.at[buf],
                                  isems[buf]).wait()

            def sbody(t, carry):
                jj = t >> 3
                m = t & 7
                v = src_v[buf, jj, pl.ds(m * 16, 16)]
                src_v[buf, jj, pl.ds(m * 16, 16)] = v * 8 + dc
                return carry

            lax.fori_loop(0, SEG * (SB // 16), sbody, None)

        def fire_chunk(buf, q, rbuf):
            for i in range(CH):
                pltpu.async_copy(x_hbm.at[src_v.at[buf, q * CH + i]],
                                 rows_v.at[rbuf, i], rsems[rbuf])

        def compute(buf, q, rbuf):
            for i in range(CH):
                pltpu.make_async_copy(x_hbm.at[pl.ds(0, SB)],
                                      rows_v.at[rbuf, i], rsems[rbuf]).wait()

            def mg_body(mg, carry):
                i = mg >> 3
                m = mg & 7
                j = q * CH + i
                e0 = m * 16
                dst16 = dst_v[buf, j, pl.ds(e0, 16)]
                row16 = dst16 - lo16
                cm = row16.astype(jnp.uint32) < jnp.uint32(HALF_N)
                plsc.addupdate_scatter(cnt_v, [row16], ones, mask=cm)
                for eu in range(16):
                    r16 = jnp.full((16,), row16[eu], jnp.int32)
                    mk = r16.astype(jnp.uint32) < jnp.uint32(HALF_N)
                    data = rows_v[rbuf, i, e0 + eu, :]
                    plsc.addupdate_scatter(acc_v, [r16, iota], data, mask=mk)
                return carry

            lax.fori_loop(0, CH * (SB // 16), mg_body, None)

        fire_idx(0, 0)
        fire_idx(1, 1)
        scale(0)
        fire_chunk(0, 0, 0)
        fire_chunk(0, 1, 1)

        def seg_pair(sp, carry):
            for b in (0, 1):
                seg = sp * 2 + b
                nb = 1 - b
                for q in range(NQ):
                    rb = (b + q) % 2
                    compute(b, q, rb)
                    t = q + 2
                    if t < NQ:
                        fire_chunk(b, t, (b + t) % 2)
                    elif t == NQ:
                        @pl.when(seg + 1 < NS)
                        def _():
                            scale(nb)
                            fire_chunk(nb, 0, rb)
                    else:
                        @pl.when(seg + 1 < NS)
                        def _():
                            fire_chunk(nb, 1, rb)

                @pl.when(seg + 2 < NS)
                def _():
                    fire_idx(seg + 2, b)
            return carry

        lax.fori_loop(0, NS // 2, seg_pair, None)

        pltpu.sync_copy(acc_v,
                        agg_hbm.at[c, pl.ds(nh * HALF_N, HALF_N),
                                   pl.ds(dc * DF, DF)])

        @pl.when(dc == 0)
        def _():
            pltpu.sync_copy(cnt_v.at[pl.ds(0, HALF_N)], cnt_hbm.at[c, nh])

    return body(xg, src2d, dst2d)


def _tc_combine(agg, cnt, x, W_l, b_l, W_r):
    def body(agg_ref, cnt_ref, x_ref, wl_ref, wr_ref, b_ref, out_ref):
        a = agg_ref[0] + agg_ref[1]
        cb = cnt_ref[0, 0] + cnt_ref[0, 1]
        inv = 1.0 / jnp.maximum(cb, 1.0)
        mean = a * inv[:, None]
        dn = (((1,), (1,)), ((), ()))
        acc = lax.dot_general(mean, wl_ref[...], dn,
                              preferred_element_type=jnp.float32)
        acc = acc + lax.dot_general(x_ref[...], wr_ref[...], dn,
                                    preferred_element_type=jnp.float32)
        out_ref[...] = acc + b_ref[...]

    return pl.pallas_call(
        body,
        grid=(GRID,),
        in_specs=[
            pl.BlockSpec((NCORE, R, D), lambda i: (0, i, 0)),
            pl.BlockSpec((1, NCORE, R), lambda i: (i, 0, 0)),
            pl.BlockSpec((R, D), lambda i: (i, 0)),
            pl.BlockSpec((D, D), lambda i: (0, 0)),
            pl.BlockSpec((D, D), lambda i: (0, 0)),
            pl.BlockSpec((1, D), lambda i: (0, 0)),
        ],
        out_specs=pl.BlockSpec((R, D), lambda i: (i, 0)),
        out_shape=jax.ShapeDtypeStruct((N, D), jnp.float32),
    )(agg, cnt, x, W_l, W_r, b_l)


def kernel(x, edge_index, W_l, b_l, W_r):
    ei = edge_index.astype(jnp.int32)
    src2d = ei[0].reshape(E // SB, SB)
    dst2d = ei[1].reshape(E // SB, SB)
    xg = x.reshape(N * 8, DF)
    agg, cnt = _sc_aggregate(xg, src2d, dst2d)
    cnt2 = cnt.reshape(NCORE, N).reshape(NCORE, GRID, R).transpose(1, 0, 2)
    return _tc_combine(agg, cnt2, x, W_l, b_l.reshape(1, D), W_r)


# speedup vs baseline: 4.9847x; 1.7164x over previous
import functools

import jax
import jax.numpy as jnp
from jax import lax
from jax.experimental import pallas as pl
from jax.experimental.pallas import tpu as pltpu
from jax.experimental.pallas import tpu_sc as plsc

N = 10000
E = 320000
D = 128
HALF_N = N // 2
DF = 16
NSUB = 16
NCORE = 2
SB = 128
SEG = 25
CH = 5
NQ = SEG // CH
ROWS_PER_CORE = E // NCORE // SB
NS = ROWS_PER_CORE // SEG
IBYTES = 2 * SEG * SB * 4
GBYTES = CH * SB * DF * 4

R = 200
GRID = N // R


def _sc_aggregate(xg, src2d, dst2d):
    mesh = plsc.VectorSubcoreMesh(core_axis_name="c", subcore_axis_name="s")

    @functools.partial(
        pl.kernel,
        out_type=[
            jax.ShapeDtypeStruct((NCORE, N, D), jnp.float32),
            jax.ShapeDtypeStruct((NCORE, 2, HALF_N), jnp.float32),
        ],
        mesh=mesh,
        compiler_params=pltpu.CompilerParams(use_tc_tiling_on_sc=False,
                                             needs_layout_passes=False),
        scratch_types=[
            pltpu.VMEM((HALF_N, DF), jnp.float32),
            pltpu.VMEM((5120,), jnp.float32),
            pltpu.VMEM((2, SEG, SB), jnp.int32),
            pltpu.VMEM((2, SEG, SB), jnp.int32),
            pltpu.VMEM((2, CH, SB, DF), jnp.float32),
            pltpu.SemaphoreType.DMA,
            pltpu.SemaphoreType.DMA,
            pltpu.SemaphoreType.DMA,
            pltpu.SemaphoreType.DMA,
        ],
    )
    def body(x_hbm, src_hbm, dst_hbm, agg_hbm, cnt_hbm,
             acc_v, cnt_v, src_v, dst_v, rows_v, isem0, isem1, rsem0, rsem1):
        c = lax.axis_index("c")
        s = lax.axis_index("s")
        nh = s // 8
        dc = s % 8
        lo16 = jnp.full((16,), nh * HALF_N, jnp.int32)
        iota = lax.iota(jnp.int32, 16)
        zeros = jnp.zeros((16,), jnp.float32)
        ones = jnp.ones((16,), jnp.float32)
        isems = (isem0, isem1)
        rsems = (rsem0, rsem1)

        def zacc(r, carry):
            acc_v[r, :] = zeros
            return carry

        lax.fori_loop(0, HALF_N, zacc, None)

        def zcnt(r, carry):
            cnt_v[pl.ds(r * 16, 16)] = zeros
            return carry

        lax.fori_loop(0, 5120 // 16, zcnt, None)

        def fire_idx(seg, buf):
            r0 = c * ROWS_PER_CORE + seg * SEG
            pltpu.async_copy(src_hbm.at[pl.ds(r0, SEG)], src_v.at[buf],
                             isems[buf])
            pltpu.async_copy(dst_hbm.at[pl.ds(r0, SEG)], dst_v.at[buf],
                             isems[buf])

        def scale(buf):
            pltpu.make_async_copy(src_hbm.at[pl.ds(0, SEG)], src_v.at[buf],
                                  isems[buf]).wait()
            pltpu.make_async_copy(dst_hbm.at[pl.ds(0, SEG)], dst_v.at[buf],
                                  isems[buf]).wait()

            def sbody(t, carry):
                jj = t >> 3
                m = t & 7
                v = src_v[buf, jj, pl.ds(m * 16, 16)]
                src_v[buf, jj, pl.ds(m * 16, 16)] = v * 8 + dc
                return carry

            lax.fori_loop(0, SEG * (SB // 16), sbody, None)

        def fire_chunk(buf, q, rbuf):
            for i in range(CH):
                pltpu.async_copy(x_hbm.at[src_v.at[buf, q * CH + i]],
                                 rows_v.at[rbuf, i], rsems[rbuf])

        def compute(buf, q, rbuf):
            for i in range(CH):
                pltpu.make_async_copy(x_hbm.at[pl.ds(0, SB)],
                                      rows_v.at[rbuf, i], rsems[rbuf]).wait()

            def mg_body(mg, carry):
                i = mg >> 3
                m = mg & 7
                j = q * CH + i
                e0 = m * 16
                dst16 = dst_v[buf, j, pl.ds(e0, 16)]
                row16 = dst16 - lo16
                cm = row16.astype(jnp.uint32) < jnp.uint32(HALF_N)
                plsc.addupdate_scatter(cnt_v, [row16], ones, mask=cm)
                idxs, msks, datas = [], [], []
                for eu in range(16):
                    r16 = jnp.full((16,), row16[eu], jnp.int32)
                    idxs.append(r16)
                    msks.append(r16.astype(jnp.uint32) < jnp.uint32(HALF_N))
                    datas.append(rows_v[rbuf, i, e0 + eu, :])
                for eu in range(16):
                    plsc.addupdate_scatter(acc_v, [idxs[eu], iota], datas[eu],
                                           mask=msks[eu])
                return carry

            lax.fori_loop(0, CH * (SB // 16), mg_body, None)

        fire_idx(0, 0)
        fire_idx(1, 1)
        scale(0)
        fire_chunk(0, 0, 0)
        fire_chunk(0, 1, 1)

        def seg_pair(sp, carry):
            for b in (0, 1):
                seg = sp * 2 + b
                nb = 1 - b
                for q in range(NQ):
                    rb = (b + q) % 2
                    compute(b, q, rb)
                    t = q + 2
                    if t < NQ:
                        fire_chunk(b, t, (b + t) % 2)
                    elif t == NQ:
                        @pl.when(seg + 1 < NS)
                        def _():
                            scale(nb)
                            fire_chunk(nb, 0, rb)
                    else:
                        @pl.when(seg + 1 < NS)
                        def _():
                            fire_chunk(nb, 1, rb)

                @pl.when(seg + 2 < NS)
                def _():
                    fire_idx(seg + 2, b)
            return carry

        lax.fori_loop(0, NS // 2, seg_pair, None)

        pltpu.sync_copy(acc_v,
                        agg_hbm.at[c, pl.ds(nh * HALF_N, HALF_N),
                                   pl.ds(dc * DF, DF)])

        @pl.when(dc == 0)
        def _():
            pltpu.sync_copy(cnt_v.at[pl.ds(0, HALF_N)], cnt_hbm.at[c, nh])

    return body(xg, src2d, dst2d)


def _tc_combine(agg, cnt, x, W_l, b_l, W_r):
    def body(agg_ref, cnt_ref, x_ref, wl_ref, wr_ref, b_ref, out_ref):
        a = agg_ref[0] + agg_ref[1]
        cb = cnt_ref[0, 0] + cnt_ref[0, 1]
        inv = 1.0 / jnp.maximum(cb, 1.0)
        mean = a * inv[:, None]
        dn = (((1,), (1,)), ((), ()))
        acc = lax.dot_general(mean, wl_ref[...], dn,
                              preferred_element_type=jnp.float32)
        acc = acc + lax.dot_general(x_ref[...], wr_ref[...], dn,
                                    preferred_element_type=jnp.float32)
        out_ref[...] = acc + b_ref[...]

    return pl.pallas_call(
        body,
        grid=(GRID,),
        in_specs=[
            pl.BlockSpec((NCORE, R, D), lambda i: (0, i, 0)),
            pl.BlockSpec((1, NCORE, R), lambda i: (i, 0, 0)),
            pl.BlockSpec((R, D), lambda i: (i, 0)),
            pl.BlockSpec((D, D), lambda i: (0, 0)),
            pl.BlockSpec((D, D), lambda i: (0, 0)),
            pl.BlockSpec((1, D), lambda i: (0, 0)),
        ],
        out_specs=pl.BlockSpec((R, D), lambda i: (i, 0)),
        out_shape=jax.ShapeDtypeStruct((N, D), jnp.float32),
    )(agg, cnt, x, W_l, W_r, b_l)


def kernel(x, edge_index, W_l, b_l, W_r):
    ei = edge_index.astype(jnp.int32)
    src2d = ei[0].reshape(E // SB, SB)
    dst2d = ei[1].reshape(E // SB, SB)
    xg = x.reshape(N * 8, DF)
    agg, cnt = _sc_aggregate(xg, src2d, dst2d)
    cnt2 = cnt.reshape(NCORE, N).reshape(NCORE, GRID, R).transpose(1, 0, 2)
    return _tc_combine(agg, cnt2, x, W_l, b_l.reshape(1, D), W_r)


# speedup vs baseline: 5.9248x; 1.1886x over previous
import functools

import jax
import jax.numpy as jnp
from jax import lax
from jax.experimental import pallas as pl
from jax.experimental.pallas import tpu as pltpu
from jax.experimental.pallas import tpu_sc as plsc

N = 10000
E = 320000
D = 128
DF = 8
NSUB = 16
NCORE = 2
SB = 128
SEG = 25
CH = 5
NQ = SEG // CH
ROWS_PER_CORE = E // NCORE // SB
NS = ROWS_PER_CORE // SEG

R = 200
GRID = N // R


def _sc_aggregate(xg, src2d, dst2d):
    mesh = plsc.VectorSubcoreMesh(core_axis_name="c", subcore_axis_name="s")

    @functools.partial(
        pl.kernel,
        out_type=[
            jax.ShapeDtypeStruct((NCORE, N, D), jnp.float32),
            jax.ShapeDtypeStruct((NCORE, N), jnp.float32),
        ],
        mesh=mesh,
        compiler_params=pltpu.CompilerParams(use_tc_tiling_on_sc=False,
                                             needs_layout_passes=False),
        scratch_types=[
            pltpu.VMEM((N, DF), jnp.float32),
            pltpu.VMEM((N,), jnp.float32),
            pltpu.VMEM((2, SEG, SB), jnp.int32),
            pltpu.VMEM((2, SEG, SB), jnp.int32),
            pltpu.VMEM((2, CH, SB, DF), jnp.float32),
            pltpu.SemaphoreType.DMA,
            pltpu.SemaphoreType.DMA,
            pltpu.SemaphoreType.DMA,
            pltpu.SemaphoreType.DMA,
        ],
    )
    def body(x_hbm, src_hbm, dst_hbm, agg_hbm, cnt_hbm,
             acc_v, cnt_v, src_v, dst_v, rows_v, isem0, isem1, rsem0, rsem1):
        c = lax.axis_index("c")
        dc = lax.axis_index("s")
        iota = lax.iota(jnp.int32, 16)
        _PAT01 = iota >> 3
        _COLPAT = iota & 7
        _PATS = [_PAT01 + 2 * p for p in range(8)]
        zeros = jnp.zeros((16,), jnp.float32)
        ones = jnp.ones((16,), jnp.float32)
        isems = (isem0, isem1)
        rsems = (rsem0, rsem1)

        def zacc(r, carry):
            for u in range(8):
                row16 = (r * 16 + 2 * u) + _PAT01
                plsc.store_scatter(acc_v, [row16, _COLPAT], zeros)
            cnt_v[pl.ds(r * 16, 16)] = zeros
            return carry

        lax.fori_loop(0, N // 16, zacc, None)

        def fire_idx(seg, buf):
            r0 = c * ROWS_PER_CORE + seg * SEG
            pltpu.async_copy(src_hbm.at[pl.ds(r0, SEG)], src_v.at[buf],
                             isems[buf])
            pltpu.async_copy(dst_hbm.at[pl.ds(r0, SEG)], dst_v.at[buf],
                             isems[buf])

        def scale(buf):
            pltpu.make_async_copy(src_hbm.at[pl.ds(0, SEG)], src_v.at[buf],
                                  isems[buf]).wait()
            pltpu.make_async_copy(dst_hbm.at[pl.ds(0, SEG)], dst_v.at[buf],
                                  isems[buf]).wait()

            def sbody(t, carry):
                jj = t >> 3
                m = t & 7
                v = src_v[buf, jj, pl.ds(m * 16, 16)]
                src_v[buf, jj, pl.ds(m * 16, 16)] = v * NSUB + dc
                return carry

            lax.fori_loop(0, SEG * (SB // 16), sbody, None)

        def fire_chunk(buf, q, rbuf):
            for i in range(CH):
                pltpu.async_copy(x_hbm.at[src_v.at[buf, q * CH + i]],
                                 rows_v.at[rbuf, i], rsems[rbuf])

        def compute(buf, q, rbuf):
            for i in range(CH):
                pltpu.make_async_copy(x_hbm.at[pl.ds(0, SB)],
                                      rows_v.at[rbuf, i], rsems[rbuf]).wait()

            def mg_body(mg, carry):
                i = mg >> 3
                m = mg & 7
                j = q * CH + i
                e0 = m * 16
                dst16 = dst_v[buf, j, pl.ds(e0, 16)]
                plsc.addupdate_scatter(cnt_v, [dst16], ones)
                idxs, datas = [], []
                for p in range(8):
                    idxs.append(dst16.at[_PATS[p]].get(
                        mode="promise_in_bounds"))
                    rpat = _PAT01 + (e0 + 2 * p)
                    datas.append(plsc.load_gather(rows_v.at[rbuf, i],
                                                  [rpat, _COLPAT]))
                for p in range(8):
                    plsc.addupdate_scatter(acc_v, [idxs[p], _COLPAT],
                                           datas[p])
                return carry

            lax.fori_loop(0, CH * (SB // 16), mg_body, None)

        fire_idx(0, 0)
        fire_idx(1, 1)
        scale(0)
        fire_chunk(0, 0, 0)
        fire_chunk(0, 1, 1)

        def seg_pair(sp, carry):
            for b in (0, 1):
                seg = sp * 2 + b
                nb = 1 - b
                for q in range(NQ):
                    rb = (b + q) % 2
                    compute(b, q, rb)
                    t = q + 2
                    if t < NQ:
                        fire_chunk(b, t, (b + t) % 2)
                    elif t == NQ:
                        @pl.when(seg + 1 < NS)
                        def _():
                            scale(nb)
                            fire_chunk(nb, 0, rb)
                    else:
                        @pl.when(seg + 1 < NS)
                        def _():
                            fire_chunk(nb, 1, rb)

                @pl.when(seg + 2 < NS)
                def _():
                    fire_idx(seg + 2, b)
            return carry

        lax.fori_loop(0, NS // 2, seg_pair, None)

        pltpu.sync_copy(acc_v, agg_hbm.at[c, :, pl.ds(dc * DF, DF)])

        @pl.when(dc == 0)
        def _():
            pltpu.sync_copy(cnt_v, cnt_hbm.at[c])

    return body(xg, src2d, dst2d)


def _tc_combine(agg, cnt, x, W_l, b_l, W_r):
    def body(agg_ref, cnt_ref, x_ref, wl_ref, wr_ref, b_ref, out_ref):
        a = agg_ref[0] + agg_ref[1]
        cb = cnt_ref[0, 0] + cnt_ref[0, 1]
        inv = 1.0 / jnp.maximum(cb, 1.0)
        mean = a * inv[:, None]
        dn = (((1,), (1,)), ((), ()))
        acc = lax.dot_general(mean, wl_ref[...], dn,
                              preferred_element_type=jnp.float32)
        acc = acc + lax.dot_general(x_ref[...], wr_ref[...], dn,
                                    preferred_element_type=jnp.float32)
        out_ref[...] = acc + b_ref[...]

    return pl.pallas_call(
        body,
        grid=(GRID,),
        in_specs=[
            pl.BlockSpec((NCORE, R, D), lambda i: (0, i, 0)),
            pl.BlockSpec((1, NCORE, R), lambda i: (i, 0, 0)),
            pl.BlockSpec((R, D), lambda i: (i, 0)),
            pl.BlockSpec((D, D), lambda i: (0, 0)),
            pl.BlockSpec((D, D), lambda i: (0, 0)),
            pl.BlockSpec((1, D), lambda i: (0, 0)),
        ],
        out_specs=pl.BlockSpec((R, D), lambda i: (i, 0)),
        out_shape=jax.ShapeDtypeStruct((N, D), jnp.float32),
    )(agg, cnt, x, W_l, W_r, b_l)


def kernel(x, edge_index, W_l, b_l, W_r):
    ei = edge_index.astype(jnp.int32)
    src2d = ei[0].reshape(E // SB, SB)
    dst2d = ei[1].reshape(E // SB, SB)
    xg = x.reshape(N * NSUB, DF)
    agg, cnt = _sc_aggregate(xg, src2d, dst2d)
    cnt2 = cnt.reshape(NCORE, GRID, R).transpose(1, 0, 2)
    return _tc_combine(agg, cnt2, x, W_l, b_l.reshape(1, D), W_r)


# speedup vs baseline: 6.3762x; 1.0762x over previous
import functools

import jax
import jax.numpy as jnp
from jax import lax
from jax.experimental import pallas as pl
from jax.experimental.pallas import tpu as pltpu
from jax.experimental.pallas import tpu_sc as plsc

N = 10000
E = 320000
D = 128
DF = 8
NSUB = 16
NCORE = 2
SB = 128
SEG = 25
CH = 5
NQ = SEG // CH
ROWS_PER_CORE = E // NCORE // SB
NS = ROWS_PER_CORE // SEG

R = 200
GRID = N // R


def _sc_aggregate(xg, src2d, dst2d):
    mesh = plsc.VectorSubcoreMesh(core_axis_name="c", subcore_axis_name="s")

    @functools.partial(
        pl.kernel,
        out_type=[
            jax.ShapeDtypeStruct((NCORE, N, D), jnp.float32),
            jax.ShapeDtypeStruct((NCORE, N), jnp.float32),
        ],
        mesh=mesh,
        compiler_params=pltpu.CompilerParams(use_tc_tiling_on_sc=False,
                                             needs_layout_passes=False),
        scratch_types=[
            pltpu.VMEM((N, DF), jnp.float32),
            pltpu.VMEM((N,), jnp.float32),
            pltpu.VMEM((2, SEG, SB), jnp.int32),
            pltpu.VMEM((2, SEG, SB), jnp.int32),
            pltpu.VMEM((2, CH, SB, DF), jnp.float32),
            pltpu.SemaphoreType.DMA,
            pltpu.SemaphoreType.DMA,
            pltpu.SemaphoreType.DMA,
            pltpu.SemaphoreType.DMA,
        ],
    )
    def body(x_hbm, src_hbm, dst_hbm, agg_hbm, cnt_hbm,
             acc_v, cnt_v, src_v, dst_v, rows_v, isem0, isem1, rsem0, rsem1):
        c = lax.axis_index("c")
        dc = lax.axis_index("s")
        iota = lax.iota(jnp.int32, 16)
        _PAT01 = iota >> 3
        _COLPAT = iota & 7
        _PATS = [_PAT01 + 2 * p for p in range(8)]
        zeros = jnp.zeros((16,), jnp.float32)
        ones = jnp.ones((16,), jnp.float32)
        isems = (isem0, isem1)
        rsems = (rsem0, rsem1)

        tbl = x_hbm.at[pl.ds(dc, N * NSUB - NSUB + 1)]

        def fire_idx(seg, buf):
            r0 = c * ROWS_PER_CORE + seg * SEG
            pltpu.async_copy(src_hbm.at[pl.ds(r0, SEG)], src_v.at[buf],
                             isems[buf])
            pltpu.async_copy(dst_hbm.at[pl.ds(r0, SEG)], dst_v.at[buf],
                             isems[buf])

        fire_idx(0, 0)
        fire_idx(1, 1)

        def zacc(r, carry):
            for u in range(8):
                row16 = (r * 16 + 2 * u) + _PAT01
                plsc.store_scatter(acc_v, [row16, _COLPAT], zeros)
            cnt_v[pl.ds(r * 16, 16)] = zeros
            return carry

        lax.fori_loop(0, N // 16, zacc, None)

        def wait_idx(buf):
            pltpu.make_async_copy(src_hbm.at[pl.ds(0, SEG)], src_v.at[buf],
                                  isems[buf]).wait()
            pltpu.make_async_copy(dst_hbm.at[pl.ds(0, SEG)], dst_v.at[buf],
                                  isems[buf]).wait()

        def fire_chunk(buf, q, rbuf):
            for i in range(CH):
                pltpu.async_copy(tbl.at[src_v.at[buf, q * CH + i]],
                                 rows_v.at[rbuf, i], rsems[rbuf])

        def compute(buf, q, rbuf):
            for i in range(CH):
                pltpu.make_async_copy(x_hbm.at[pl.ds(0, SB)],
                                      rows_v.at[rbuf, i], rsems[rbuf]).wait()

            def mg_body(mg, carry):
                i = mg >> 3
                m = mg & 7
                j = q * CH + i
                e0 = m * 16
                dst16 = dst_v[buf, j, pl.ds(e0, 16)]
                plsc.addupdate_scatter(cnt_v, [dst16], ones)
                idxs, datas = [], []
                for p in range(8):
                    idxs.append(dst16.at[_PATS[p]].get(
                        mode="promise_in_bounds"))
                    rpat = _PAT01 + (e0 + 2 * p)
                    datas.append(plsc.load_gather(rows_v.at[rbuf, i],
                                                  [rpat, _COLPAT]))
                for p in range(8):
                    plsc.addupdate_scatter(acc_v, [idxs[p], _COLPAT],
                                           datas[p])
                return carry

            lax.fori_loop(0, CH * (SB // 16), mg_body, None)

        wait_idx(0)
        fire_chunk(0, 0, 0)
        fire_chunk(0, 1, 1)

        def seg_pair(sp, carry):
            for b in (0, 1):
                seg = sp * 2 + b
                nb = 1 - b
                for q in range(NQ):
                    rb = (b + q) % 2
                    compute(b, q, rb)
                    t = q + 2
                    if t < NQ:
                        fire_chunk(b, t, (b + t) % 2)
                    elif t == NQ:
                        @pl.when(seg + 1 < NS)
                        def _():
                            wait_idx(nb)
                            fire_chunk(nb, 0, rb)
                    else:
                        @pl.when(seg + 1 < NS)
                        def _():
                            fire_chunk(nb, 1, rb)

                @pl.when(seg + 2 < NS)
                def _():
                    fire_idx(seg + 2, b)
            return carry

        lax.fori_loop(0, NS // 2, seg_pair, None)

        pltpu.sync_copy(acc_v, agg_hbm.at[c, :, pl.ds(dc * DF, DF)])

        @pl.when(dc == 0)
        def _():
            pltpu.sync_copy(cnt_v, cnt_hbm.at[c])

    return body(xg, src2d, dst2d)


def _tc_combine(agg, cnt, x, W_l, b_l, W_r):
    def body(agg_ref, cnt_ref, x_ref, wl_ref, wr_ref, b_ref, out_ref):
        a = agg_ref[0] + agg_ref[1]
        cb = cnt_ref[0, 0] + cnt_ref[0, 1]
        inv = 1.0 / jnp.maximum(cb, 1.0)
        mean = a * inv[:, None]
        dn = (((1,), (1,)), ((), ()))
        acc = lax.dot_general(mean, wl_ref[...], dn,
                              preferred_element_type=jnp.float32)
        acc = acc + lax.dot_general(x_ref[...], wr_ref[...], dn,
                                    preferred_element_type=jnp.float32)
        out_ref[...] = acc + b_ref[...]

    return pl.pallas_call(
        body,
        grid=(GRID,),
        in_specs=[
            pl.BlockSpec((NCORE, R, D), lambda i: (0, i, 0)),
            pl.BlockSpec((1, NCORE, R), lambda i: (i, 0, 0)),
            pl.BlockSpec((R, D), lambda i: (i, 0)),
            pl.BlockSpec((D, D), lambda i: (0, 0)),
            pl.BlockSpec((D, D), lambda i: (0, 0)),
            pl.BlockSpec((1, D), lambda i: (0, 0)),
        ],
        out_specs=pl.BlockSpec((R, D), lambda i: (i, 0)),
        out_shape=jax.ShapeDtypeStruct((N, D), jnp.float32),
    )(agg, cnt, x, W_l, W_r, b_l)


def kernel(x, edge_index, W_l, b_l, W_r):
    ei = edge_index.astype(jnp.int32)
    src2d = (ei[0] * NSUB).reshape(E // SB, SB)
    dst2d = ei[1].reshape(E // SB, SB)
    xg = x.reshape(N * NSUB, DF)
    agg, cnt = _sc_aggregate(xg, src2d, dst2d)
    cnt2 = cnt.reshape(NCORE, GRID, R).transpose(1, 0, 2)
    return _tc_combine(agg, cnt2, x, W_l, b_l.reshape(1, D), W_r)
